# baseline (speedup 1.0000x reference)
import jax
import jax.numpy as jnp
from jax import lax
from jax.experimental import pallas as pl
from jax.experimental.pallas import tpu as pltpu
from jax.experimental.pallas import tpu_sc as plsc

N_ROW_BLK = 1000
N_NODES = 10000
N_EDGES = 320000
D = 128
NC = 2
NS = 16
NW = NC * NS
CHUNK = 64
NGRP = CHUNK // 16
IDX_ROWS = N_EDGES // CHUNK
IDX_PAD = 8
NPAD = 10240
NSLICE = NPAD // NS


def _enc_body(x_ref, pos_ref, W0a_ref, W0b_ref, b0_ref, W1_ref, b1_ref,
              W2_ref, b2_ref, Wl_ref, bl_ref, Wr_ref, br_ref,
              xl_ref, xr_ref):
    x = x_ref[...]
    pos = pos_ref[...]
    h = x @ W0a_ref[...] + pos @ W0b_ref[...] + b0_ref[...]
    h = jnp.maximum(h, 0.0)
    h = jnp.maximum(h @ W1_ref[...] + b1_ref[...], 0.0)
    h = h @ W2_ref[...] + b2_ref[...]
    xl_ref[...] = h @ Wl_ref[...] + bl_ref[...]
    xr_ref[...] = h @ Wr_ref[...] + br_ref[...]


def _encoder(x, pos, W0, b0, W1, b1, W2, b2, Wl, bl, Wr, br):
    n = x.shape[0]
    grid = (n // N_ROW_BLK,)
    row = lambda i: (i, 0)
    rep = lambda i: (0, 0)
    out_shape = [jax.ShapeDtypeStruct((n, 128), jnp.float32)] * 2
    return pl.pallas_call(
        _enc_body,
        grid=grid,
        in_specs=[
            pl.BlockSpec((N_ROW_BLK, 128), row),
            pl.BlockSpec((N_ROW_BLK, 2), row),
            pl.BlockSpec((128, 512), rep),
            pl.BlockSpec((2, 512), rep),
            pl.BlockSpec((512,), lambda i: (0,)),
            pl.BlockSpec((512, 256), rep),
            pl.BlockSpec((256,), lambda i: (0,)),
            pl.BlockSpec((256, 128), rep),
            pl.BlockSpec((128,), lambda i: (0,)),
            pl.BlockSpec((128, 128), rep),
            pl.BlockSpec((128,), lambda i: (0,)),
            pl.BlockSpec((128, 128), rep),
            pl.BlockSpec((128,), lambda i: (0,)),
        ],
        out_specs=[pl.BlockSpec((N_ROW_BLK, 128), row)] * 2,
        out_shape=out_shape,
    )(x, pos, W0[:128], W0[128:], b0, W1, b1, W2, b2, Wl, bl, Wr, br)


def _edge_body(xl_hbm, xr_hbm, src_hbm, dst_hbm, dst4_hbm, attr_hbm, vecs_hbm,
               num_hbm, den_hbm,
               acc, dacc, srci, dsti, dsti4, attrv, A0, B0, A1, B1, W, exb,
               vecs, ga0, gb0, ga1, gb1, dsem, ssem0, ssem1, ssem2, ssem3):
    cid = lax.axis_index("c")
    sid = lax.axis_index("s")
    w = cid * NS + sid
    start = (w * IDX_ROWS) // NW
    end = ((w + 1) * IDX_ROWS) // NW
    nch = end - start

    zeros16 = jnp.zeros((16,), jnp.float32)
    rows16 = lax.iota(jnp.int32, 16)

    def _zrow(i, _):
        def _zcol(jj, _):
            W[i, pl.ds(jj * 16, 16)] = zeros16
            return 0
        return lax.fori_loop(0, D // 16, _zcol, 0)
    lax.fori_loop(0, CHUNK, _zrow, 0)
    for g in range(NGRP):
        exb[pl.ds(g * 16, 16)] = zeros16
    nbase = sid * NSLICE

    def _zcp(k, _):
        pltpu.sync_copy(W, acc.at[pl.ds(nbase + k * CHUNK, CHUNK)])
        pltpu.sync_copy(exb, dacc.at[pl.ds(nbase + k * CHUNK, CHUNK)])
        return 0
    lax.fori_loop(0, NSLICE // CHUNK, _zcp, 0)

    pltpu.sync_copy(vecs_hbm, vecs)
    we8 = [vecs[0, pl.ds(k * 16, 16)] for k in range(8)]
    at8 = [vecs[1, pl.ds(k * 16, 16)] for k in range(8)]
    plsc.subcore_barrier()

    def _issue(jb, bufA, bufB, semA, semB):
        a = pltpu.async_copy(xl_hbm.at[srci.at[jb]], bufA, semA)
        b = pltpu.async_copy(xr_hbm.at[dsti.at[jb]], bufB, semB)
        return a, b

    def _compute(i, jb, bufA, bufB, semA, semB):
        pltpu.make_async_copy(xl_hbm.at[srci.at[jb]], bufA, semA).wait()
        pltpu.make_async_copy(xr_hbm.at[dsti.at[jb]], bufB, semB).wait()

        jbs = jnp.full((16,), jb, jnp.int32)

        gsems = (ssem0, ssem1, ssem2, ssem3)
        for g in range(NGRP):
            @pl.when(jb != 0)
            def _():
                pltpu.make_async_copy(W.at[pl.ds(g * 16, 16)],
                                      acc.at[dsti4.at[0]], gsems[g]).wait()

            if g == 0:
                @pl.when(jb != 0)
                def _():
                    pltpu.make_async_copy(exb, dacc.at[dsti.at[jb]],
                                          dsem).wait()
            av = plsc.load_gather(attrv, [jbs, rows16 + g * 16])

            @plsc.parallel_loop(0, 16, unroll=3)
            def _edge(e):
                ge = e + g * 16
                ab = av.at[jnp.full((16,), e, jnp.int32)].get(
                    mode="promise_in_bounds")
                arow = [bufA[ge, pl.ds(k * 16, 16)] for k in range(8)]
                accs = []
                for k in range(8):
                    m = arow[k] + bufB[ge, pl.ds(k * 16, 16)] + ab * we8[k]
                    m = jnp.where(m > 0, m, 0.2 * m)
                    accs.append(m * at8[k])
                vacc = (((accs[0] + accs[1]) + (accs[2] + accs[3]))
                        + ((accs[4] + accs[5]) + (accs[6] + accs[7])))
                exv = jnp.exp(jnp.full((16,), jnp.sum(vacc)))
                for k in range(8):
                    W[ge, pl.ds(k * 16, 16)] = arow[k] * exv
                plsc.store_scatter(exb, [jnp.full((16,), ge, jnp.int32)],
                                   exv, mask=rows16 == e)

            pltpu.async_copy(W.at[pl.ds(g * 16, 16)],
                             acc.at[dsti4.at[jb * 4 + g]], gsems[g],
                             add=True)
        pltpu.async_copy(exb, dacc.at[dsti.at[jb]], dsem, add=True)

    def _chunk(i, _):
        r = start + i
        jb = i % 8

        @pl.when(jb == 0)
        def _():
            @pl.when(i > 0)
            def _():
                for g, sg in enumerate((ssem0, ssem1, ssem2, ssem3)):
                    pltpu.make_async_copy(
                        W.at[pl.ds(g * 16, 16)],
                        acc.at[dsti4.at[0]], sg).wait()
                pltpu.make_async_copy(exb, dacc.at[dsti.at[0]], dsem).wait()

            pltpu.sync_copy(src_hbm.at[pl.ds(r, 8)], srci)
            pltpu.sync_copy(dst_hbm.at[pl.ds(r, 8)], dsti)
            pltpu.sync_copy(dst4_hbm.at[pl.ds(r * 4, 32)], dsti4)
            pltpu.sync_copy(attr_hbm.at[pl.ds(r, 8)], attrv)

            @pl.when(i % 2 == 0)
            def _():
                _issue(jb, A0, B0, ga0, gb0)

            @pl.when(i % 2 == 1)
            def _():
                _issue(jb, A1, B1, ga1, gb1)

        @pl.when((i + 1 < nch) & (jb != 7))
        def _():
            @pl.when(i % 2 == 0)
            def _():
                _issue(jb + 1, A1, B1, ga1, gb1)

            @pl.when(i % 2 == 1)
            def _():
                _issue(jb + 1, A0, B0, ga0, gb0)

        @pl.when(i % 2 == 0)
        def _():
            _compute(i, jb, A0, B0, ga0, gb0)

        @pl.when(i % 2 == 1)
        def _():
            _compute(i, jb, A1, B1, ga1, gb1)
        return 0

    lax.fori_loop(0, nch, _chunk, 0)

    for g, sg in enumerate((ssem0, ssem1, ssem2, ssem3)):
        pltpu.make_async_copy(W.at[pl.ds(g * 16, 16)],
                              acc.at[dsti4.at[0]], sg).wait()
    pltpu.make_async_copy(exb, dacc.at[dsti.at[0]], dsem).wait()

    plsc.subcore_barrier()

    def _ocp(k, _):
        pltpu.sync_copy(acc.at[pl.ds(nbase + k * 128, 128)],
                        num_hbm.at[cid, pl.ds(nbase + k * 128, 128)])
        return 0
    lax.fori_loop(0, NSLICE // 128, _ocp, 0)
    pltpu.sync_copy(dacc.at[pl.ds(nbase, NSLICE)],
                    den_hbm.at[cid, pl.ds(nbase, NSLICE)])


def _edge_phase(xl, xr, src2d, dst2d, dst4, attr2d, vecs):
    mesh = plsc.VectorSubcoreMesh(core_axis_name="c", subcore_axis_name="s")
    f = pl.kernel(
        _edge_body,
        out_type=[jax.ShapeDtypeStruct((NC, NPAD, D), jnp.float32),
                  jax.ShapeDtypeStruct((NC, NPAD), jnp.float32)],
        mesh=mesh,
        compiler_params=pltpu.CompilerParams(
            needs_layout_passes=False, use_tc_tiling_on_sc=False),
        scratch_types=[
            pltpu.VMEM_SHARED((NPAD, D), jnp.float32),
            pltpu.VMEM_SHARED((NPAD,), jnp.float32),
            pltpu.VMEM((8, CHUNK), jnp.int32),
            pltpu.VMEM((8, CHUNK), jnp.int32),
            pltpu.VMEM((32, 16), jnp.int32),
            pltpu.VMEM((8, CHUNK), jnp.float32),
            pltpu.VMEM((CHUNK, D), jnp.float32),
            pltpu.VMEM((CHUNK, D), jnp.float32),
            pltpu.VMEM((CHUNK, D), jnp.float32),
            pltpu.VMEM((CHUNK, D), jnp.float32),
            pltpu.VMEM((CHUNK, D), jnp.float32),
            pltpu.VMEM((CHUNK,), jnp.float32),
            pltpu.VMEM((2, D), jnp.float32),
            pltpu.SemaphoreType.DMA,
            pltpu.SemaphoreType.DMA,
            pltpu.SemaphoreType.DMA,
            pltpu.SemaphoreType.DMA,
            pltpu.SemaphoreType.DMA,
            pltpu.SemaphoreType.DMA,
            pltpu.SemaphoreType.DMA,
            pltpu.SemaphoreType.DMA,
            pltpu.SemaphoreType.DMA,
        ],
    )
    return f(xl, xr, src2d, dst2d, dst4, attr2d, vecs)


def _dec_body(u_ref, dn_ref, bias_ref, Wd0_ref, bd0_ref, Wd1_ref, bd1_ref,
              out_ref):
    num = u_ref[0] + u_ref[1]
    den = dn_ref[0, :, :] + dn_ref[1, :, :]
    agg = num / (den + 1e-16) + bias_ref[...]
    z = jnp.maximum(agg, 0.0)
    d = jnp.maximum(z @ Wd0_ref[...] + bd0_ref[...], 0.0)
    logits = d @ Wd1_ref[...] + bd1_ref[...]
    out_ref[...] = jax.nn.softmax(logits, axis=-1)


def _decoder(u, dens, bias_g, Wd0, bd0, Wd1, bd1):
    n = N_NODES
    grid = (n // N_ROW_BLK,)
    rep = lambda i: (0, 0)
    return pl.pallas_call(
        _dec_body,
        grid=grid,
        in_specs=[
            pl.BlockSpec((NC, N_ROW_BLK, D), lambda i: (0, i, 0)),
            pl.BlockSpec((NC, N_ROW_BLK, 1), lambda i: (0, i, 0)),
            pl.BlockSpec((128,), lambda i: (0,)),
            pl.BlockSpec((128, 64), rep),
            pl.BlockSpec((64,), lambda i: (0,)),
            pl.BlockSpec((64, 30), rep),
            pl.BlockSpec((30,), lambda i: (0,)),
        ],
        out_specs=pl.BlockSpec((N_ROW_BLK, 30), lambda i: (i, 0)),
        out_shape=jax.ShapeDtypeStruct((n, 30), jnp.float32),
    )(u, dens[..., None], bias_g, Wd0, bd0, Wd1, bd1)


def kernel(x, edge_index, edge_attr, pos, W0, b0, W1, b1, W2, b2, Wl, bl,
           Wr, br, We, att, bias_g, Wd0, bd0, Wd1, bd1):
    xl, xr = _encoder(x, pos, W0, b0, W1, b1, W2, b2, Wl, bl, Wr, br)
    pad = ((0, IDX_PAD), (0, 0))
    src2d = jnp.pad(edge_index[0].reshape(IDX_ROWS, CHUNK), pad)
    dst2d = jnp.pad(edge_index[1].reshape(IDX_ROWS, CHUNK), pad)
    attr2d = jnp.pad(edge_attr.reshape(IDX_ROWS, CHUNK), pad)
    vecs = jnp.stack([We[0], att])
    dst4 = jnp.pad(edge_index[1].reshape(IDX_ROWS * 4, 16),
                   ((0, IDX_PAD * 4), (0, 0)))
    u, dens = _edge_phase(xl, xr, src2d, dst2d, dst4, attr2d, vecs)
    return _decoder(u, dens, bias_g, Wd0, bd0, Wd1, bd1)


# speedup vs baseline: 1.2271x; 1.2271x over previous
import jax
import jax.numpy as jnp
from jax import lax
from jax.experimental import pallas as pl
from jax.experimental.pallas import tpu as pltpu
from jax.experimental.pallas import tpu_sc as plsc

N_ROW_BLK = 1000
N_NODES = 10000
N_EDGES = 320000
D = 128
NC = 2
NS = 16
NW = NC * NS
CHUNK = 64
NGRP = CHUNK // 16
IDX_ROWS = N_EDGES // CHUNK
IDX_PAD = 8
NPAD = 10240
NSLICE = NPAD // NS


def _enc_body(x_ref, pos_ref, W0a_ref, W0b_ref, b0_ref, W1_ref, b1_ref,
              W2_ref, b2_ref, Wl_ref, bl_ref, Wr_ref, br_ref,
              xl_ref, xr_ref):
    x = x_ref[...]
    pos = pos_ref[...]
    h = x @ W0a_ref[...] + pos @ W0b_ref[...] + b0_ref[...]
    h = jnp.maximum(h, 0.0)
    h = jnp.maximum(h @ W1_ref[...] + b1_ref[...], 0.0)
    h = h @ W2_ref[...] + b2_ref[...]
    xl_ref[...] = h @ Wl_ref[...] + bl_ref[...]
    xr_ref[...] = h @ Wr_ref[...] + br_ref[...]


def _encoder(x, pos, W0, b0, W1, b1, W2, b2, Wl, bl, Wr, br):
    n = x.shape[0]
    grid = (n // N_ROW_BLK,)
    row = lambda i: (i, 0)
    rep = lambda i: (0, 0)
    out_shape = [jax.ShapeDtypeStruct((n, 128), jnp.float32)] * 2
    return pl.pallas_call(
        _enc_body,
        grid=grid,
        in_specs=[
            pl.BlockSpec((N_ROW_BLK, 128), row),
            pl.BlockSpec((N_ROW_BLK, 2), row),
            pl.BlockSpec((128, 512), rep),
            pl.BlockSpec((2, 512), rep),
            pl.BlockSpec((512,), lambda i: (0,)),
            pl.BlockSpec((512, 256), rep),
            pl.BlockSpec((256,), lambda i: (0,)),
            pl.BlockSpec((256, 128), rep),
            pl.BlockSpec((128,), lambda i: (0,)),
            pl.BlockSpec((128, 128), rep),
            pl.BlockSpec((128,), lambda i: (0,)),
            pl.BlockSpec((128, 128), rep),
            pl.BlockSpec((128,), lambda i: (0,)),
        ],
        out_specs=[pl.BlockSpec((N_ROW_BLK, 128), row)] * 2,
        out_shape=out_shape,
    )(x, pos, W0[:128], W0[128:], b0, W1, b1, W2, b2, Wl, bl, Wr, br)


def _edge_body(xl_hbm, xr_hbm, src_hbm, dst_hbm, dst4_hbm, attr_hbm, vecs_hbm,
               num_hbm, den_hbm,
               acc, dacc, srci, dsti, dsti4, attrv, A0, B0, A1, B1, W, exb,
               S, vecs, ga0, gb0, ga1, gb1, dsem, ssem0, ssem1, ssem2, ssem3):
    cid = lax.axis_index("c")
    sid = lax.axis_index("s")
    w = cid * NS + sid
    start = (w * IDX_ROWS) // NW
    end = ((w + 1) * IDX_ROWS) // NW
    nch = end - start

    zeros16 = jnp.zeros((16,), jnp.float32)
    rows16 = lax.iota(jnp.int32, 16)

    def _zrow(i, _):
        def _zcol(jj, _):
            W[i, pl.ds(jj * 16, 16)] = zeros16
            return 0
        return lax.fori_loop(0, D // 16, _zcol, 0)
    lax.fori_loop(0, CHUNK, _zrow, 0)
    for g in range(NGRP):
        exb[pl.ds(g * 16, 16)] = zeros16
    nbase = sid * NSLICE

    def _zcp(k, _):
        pltpu.sync_copy(W, acc.at[pl.ds(nbase + k * CHUNK, CHUNK)])
        pltpu.sync_copy(exb, dacc.at[pl.ds(nbase + k * CHUNK, CHUNK)])
        return 0
    lax.fori_loop(0, NSLICE // CHUNK, _zcp, 0)

    pltpu.sync_copy(vecs_hbm, vecs)
    we8 = [vecs[0, pl.ds(k * 16, 16)] for k in range(8)]
    at8 = [vecs[1, pl.ds(k * 16, 16)] for k in range(8)]
    plsc.subcore_barrier()

    def _issue(jb, bufA, bufB, semA, semB):
        a = pltpu.async_copy(xl_hbm.at[srci.at[jb]], bufA, semA)
        b = pltpu.async_copy(xr_hbm.at[dsti.at[jb]], bufB, semB)
        return a, b

    def _compute(i, jb, bufA, bufB, semA, semB):
        pltpu.make_async_copy(xl_hbm.at[srci.at[jb]], bufA, semA).wait()
        pltpu.make_async_copy(xr_hbm.at[dsti.at[jb]], bufB, semB).wait()

        jbs = jnp.full((16,), jb, jnp.int32)

        gsems = (ssem0, ssem1, ssem2, ssem3)
        for g in range(NGRP):
            @pl.when(jb != 0)
            def _():
                pltpu.make_async_copy(W.at[pl.ds(g * 16, 16)],
                                      acc.at[dsti4.at[0]], gsems[g]).wait()

            if g == 0:
                @pl.when(jb != 0)
                def _():
                    pltpu.make_async_copy(exb, dacc.at[dsti.at[jb]],
                                          dsem).wait()
            av = plsc.load_gather(attrv, [jbs, rows16 + g * 16])

            @plsc.parallel_loop(0, 16, unroll=2)
            def _logit(e):
                ge = e + g * 16
                ab = av.at[jnp.full((16,), e, jnp.int32)].get(
                    mode="promise_in_bounds")
                acc0 = zeros16
                acc1 = zeros16
                for k in range(8):
                    m = (bufA[ge, pl.ds(k * 16, 16)]
                         + bufB[ge, pl.ds(k * 16, 16)] + ab * we8[k])
                    m = jnp.where(m > 0, m, 0.2 * m)
                    if k % 2 == 0:
                        acc0 = acc0 + m * at8[k]
                    else:
                        acc1 = acc1 + m * at8[k]
                S[e, pl.ds(0, 16)] = jnp.full((16,), jnp.sum(acc0 + acc1))

            lv = plsc.load_gather(S, [rows16, jnp.zeros((16,), jnp.int32)])
            ex16 = jnp.exp(lv)
            exb[pl.ds(g * 16, 16)] = ex16

            @plsc.parallel_loop(0, 16, unroll=2)
            def _scale(e):
                ge = e + g * 16
                exv = ex16.at[jnp.full((16,), e, jnp.int32)].get(
                    mode="promise_in_bounds")
                for k in range(8):
                    W[ge, pl.ds(k * 16, 16)] = bufA[ge, pl.ds(k * 16, 16)] * exv

            pltpu.async_copy(W.at[pl.ds(g * 16, 16)],
                             acc.at[dsti4.at[jb * 4 + g]], gsems[g],
                             add=True)
        pltpu.async_copy(exb, dacc.at[dsti.at[jb]], dsem, add=True)

    def _chunk(i, _):
        r = start + i
        jb = i % 8

        @pl.when(jb == 0)
        def _():
            @pl.when(i > 0)
            def _():
                for g, sg in enumerate((ssem0, ssem1, ssem2, ssem3)):
                    pltpu.make_async_copy(
                        W.at[pl.ds(g * 16, 16)],
                        acc.at[dsti4.at[0]], sg).wait()
                pltpu.make_async_copy(exb, dacc.at[dsti.at[0]], dsem).wait()

            pltpu.sync_copy(src_hbm.at[pl.ds(r, 8)], srci)
            pltpu.sync_copy(dst_hbm.at[pl.ds(r, 8)], dsti)
            pltpu.sync_copy(dst4_hbm.at[pl.ds(r * 4, 32)], dsti4)
            pltpu.sync_copy(attr_hbm.at[pl.ds(r, 8)], attrv)

            @pl.when(i % 2 == 0)
            def _():
                _issue(jb, A0, B0, ga0, gb0)

            @pl.when(i % 2 == 1)
            def _():
                _issue(jb, A1, B1, ga1, gb1)

        @pl.when((i + 1 < nch) & (jb != 7))
        def _():
            @pl.when(i % 2 == 0)
            def _():
                _issue(jb + 1, A1, B1, ga1, gb1)

            @pl.when(i % 2 == 1)
            def _():
                _issue(jb + 1, A0, B0, ga0, gb0)

        @pl.when(i % 2 == 0)
        def _():
            _compute(i, jb, A0, B0, ga0, gb0)

        @pl.when(i % 2 == 1)
        def _():
            _compute(i, jb, A1, B1, ga1, gb1)
        return 0

    lax.fori_loop(0, nch, _chunk, 0)

    for g, sg in enumerate((ssem0, ssem1, ssem2, ssem3)):
        pltpu.make_async_copy(W.at[pl.ds(g * 16, 16)],
                              acc.at[dsti4.at[0]], sg).wait()
    pltpu.make_async_copy(exb, dacc.at[dsti.at[0]], dsem).wait()

    plsc.subcore_barrier()

    def _ocp(k, _):
        pltpu.sync_copy(acc.at[pl.ds(nbase + k * 128, 128)],
                        num_hbm.at[cid, pl.ds(nbase + k * 128, 128)])
        return 0
    lax.fori_loop(0, NSLICE // 128, _ocp, 0)
    pltpu.sync_copy(dacc.at[pl.ds(nbase, NSLICE)],
                    den_hbm.at[cid, pl.ds(nbase, NSLICE)])


def _edge_phase(xl, xr, src2d, dst2d, dst4, attr2d, vecs):
    mesh = plsc.VectorSubcoreMesh(core_axis_name="c", subcore_axis_name="s")
    f = pl.kernel(
        _edge_body,
        out_type=[jax.ShapeDtypeStruct((NC, NPAD, D), jnp.float32),
                  jax.ShapeDtypeStruct((NC, NPAD), jnp.float32)],
        mesh=mesh,
        compiler_params=pltpu.CompilerParams(
            needs_layout_passes=False, use_tc_tiling_on_sc=False),
        scratch_types=[
            pltpu.VMEM_SHARED((NPAD, D), jnp.float32),
            pltpu.VMEM_SHARED((NPAD,), jnp.float32),
            pltpu.VMEM((8, CHUNK), jnp.int32),
            pltpu.VMEM((8, CHUNK), jnp.int32),
            pltpu.VMEM((32, 16), jnp.int32),
            pltpu.VMEM((8, CHUNK), jnp.float32),
            pltpu.VMEM((CHUNK, D), jnp.float32),
            pltpu.VMEM((CHUNK, D), jnp.float32),
            pltpu.VMEM((CHUNK, D), jnp.float32),
            pltpu.VMEM((CHUNK, D), jnp.float32),
            pltpu.VMEM((CHUNK, D), jnp.float32),
            pltpu.VMEM((CHUNK,), jnp.float32),
            pltpu.VMEM((16, 17), jnp.float32),
            pltpu.VMEM((2, D), jnp.float32),
            pltpu.SemaphoreType.DMA,
            pltpu.SemaphoreType.DMA,
            pltpu.SemaphoreType.DMA,
            pltpu.SemaphoreType.DMA,
            pltpu.SemaphoreType.DMA,
            pltpu.SemaphoreType.DMA,
            pltpu.SemaphoreType.DMA,
            pltpu.SemaphoreType.DMA,
            pltpu.SemaphoreType.DMA,
        ],
    )
    return f(xl, xr, src2d, dst2d, dst4, attr2d, vecs)


def _dec_body(u_ref, dn_ref, bias_ref, Wd0_ref, bd0_ref, Wd1_ref, bd1_ref,
              out_ref):
    num = u_ref[0] + u_ref[1]
    den = dn_ref[0, :, :] + dn_ref[1, :, :]
    agg = num / (den + 1e-16) + bias_ref[...]
    z = jnp.maximum(agg, 0.0)
    d = jnp.maximum(z @ Wd0_ref[...] + bd0_ref[...], 0.0)
    logits = d @ Wd1_ref[...] + bd1_ref[...]
    out_ref[...] = jax.nn.softmax(logits, axis=-1)


def _decoder(u, dens, bias_g, Wd0, bd0, Wd1, bd1):
    n = N_NODES
    grid = (n // N_ROW_BLK,)
    rep = lambda i: (0, 0)
    return pl.pallas_call(
        _dec_body,
        grid=grid,
        in_specs=[
            pl.BlockSpec((NC, N_ROW_BLK, D), lambda i: (0, i, 0)),
            pl.BlockSpec((NC, N_ROW_BLK, 1), lambda i: (0, i, 0)),
            pl.BlockSpec((128,), lambda i: (0,)),
            pl.BlockSpec((128, 64), rep),
            pl.BlockSpec((64,), lambda i: (0,)),
            pl.BlockSpec((64, 30), rep),
            pl.BlockSpec((30,), lambda i: (0,)),
        ],
        out_specs=pl.BlockSpec((N_ROW_BLK, 30), lambda i: (i, 0)),
        out_shape=jax.ShapeDtypeStruct((n, 30), jnp.float32),
    )(u, dens[..., None], bias_g, Wd0, bd0, Wd1, bd1)


def kernel(x, edge_index, edge_attr, pos, W0, b0, W1, b1, W2, b2, Wl, bl,
           Wr, br, We, att, bias_g, Wd0, bd0, Wd1, bd1):
    xl, xr = _encoder(x, pos, W0, b0, W1, b1, W2, b2, Wl, bl, Wr, br)
    pad = ((0, IDX_PAD), (0, 0))
    src2d = jnp.pad(edge_index[0].reshape(IDX_ROWS, CHUNK), pad)
    dst2d = jnp.pad(edge_index[1].reshape(IDX_ROWS, CHUNK), pad)
    attr2d = jnp.pad(edge_attr.reshape(IDX_ROWS, CHUNK), pad)
    vecs = jnp.stack([We[0], att])
    dst4 = jnp.pad(edge_index[1].reshape(IDX_ROWS * 4, 16),
                   ((0, IDX_PAD * 4), (0, 0)))
    u, dens = _edge_phase(xl, xr, src2d, dst2d, dst4, attr2d, vecs)
    return _decoder(u, dens, bias_g, Wd0, bd0, Wd1, bd1)


# speedup vs baseline: 1.5444x; 1.2586x over previous
import jax
import jax.numpy as jnp
from jax import lax
from jax.experimental import pallas as pl
from jax.experimental.pallas import tpu as pltpu
from jax.experimental.pallas import tpu_sc as plsc

N_ROW_BLK = 1000
N_NODES = 10000
N_EDGES = 320000
D = 128
NC = 2
NS = 16
NW = NC * NS
CHUNK = 64
NGRP = CHUNK // 16
IDX_ROWS = N_EDGES // CHUNK
IDX_PAD = 16
NPAD = 10000
NSLICE = NPAD // NS
DPAD = 10240
DSLICE = DPAD // NS


def _enc_body(x_ref, pos_ref, W0a_ref, W0b_ref, b0_ref, W1_ref, b1_ref,
              W2_ref, b2_ref, Wl_ref, bl_ref, Wr_ref, br_ref,
              xl_ref, xr_ref):
    x = x_ref[...]
    pos = pos_ref[...]
    h = x @ W0a_ref[...] + pos @ W0b_ref[...] + b0_ref[...]
    h = jnp.maximum(h, 0.0)
    h = jnp.maximum(h @ W1_ref[...] + b1_ref[...], 0.0)
    h = h @ W2_ref[...] + b2_ref[...]
    xl_ref[...] = h @ Wl_ref[...] + bl_ref[...]
    xr_ref[...] = h @ Wr_ref[...] + br_ref[...]


def _encoder(x, pos, W0, b0, W1, b1, W2, b2, Wl, bl, Wr, br):
    n = x.shape[0]
    grid = (n // N_ROW_BLK,)
    row = lambda i: (i, 0)
    rep = lambda i: (0, 0)
    out_shape = [jax.ShapeDtypeStruct((n, 128), jnp.float32)] * 2
    return pl.pallas_call(
        _enc_body,
        grid=grid,
        in_specs=[
            pl.BlockSpec((N_ROW_BLK, 128), row),
            pl.BlockSpec((N_ROW_BLK, 2), row),
            pl.BlockSpec((128, 512), rep),
            pl.BlockSpec((2, 512), rep),
            pl.BlockSpec((512,), lambda i: (0,)),
            pl.BlockSpec((512, 256), rep),
            pl.BlockSpec((256,), lambda i: (0,)),
            pl.BlockSpec((256, 128), rep),
            pl.BlockSpec((128,), lambda i: (0,)),
            pl.BlockSpec((128, 128), rep),
            pl.BlockSpec((128,), lambda i: (0,)),
            pl.BlockSpec((128, 128), rep),
            pl.BlockSpec((128,), lambda i: (0,)),
        ],
        out_specs=[pl.BlockSpec((N_ROW_BLK, 128), row)] * 2,
        out_shape=out_shape,
    )(x, pos, W0[:128], W0[128:], b0, W1, b1, W2, b2, Wl, bl, Wr, br)


def _edge_body(xl_hbm, xr_hbm, src_hbm, dst_hbm, dst4_hbm, attr_hbm, vecs_hbm,
               num_hbm, den_hbm,
               acc, dacc, srci, dsti, dsti4, attrv, A0, B0, A1, B1, W, exb,
               vecs, ga0, gb0, ga1, gb1, dsem, ssem0, ssem1, ssem2, ssem3):
    cid = lax.axis_index("c")
    sid = lax.axis_index("s")
    w = cid * NS + sid
    start = (w * IDX_ROWS) // NW
    end = ((w + 1) * IDX_ROWS) // NW
    nch = end - start

    zeros16 = jnp.zeros((16,), jnp.float32)
    rows16 = lax.iota(jnp.int32, 16)

    def _zrow(i, _):
        def _zcol(jj, _):
            W[i, pl.ds(jj * 16, 16)] = zeros16
            return 0
        return lax.fori_loop(0, D // 16, _zcol, 0)
    lax.fori_loop(0, CHUNK, _zrow, 0)
    for g in range(NGRP):
        exb[pl.ds(g * 16, 16)] = zeros16
    nbase = sid * NSLICE

    def _zcp(k, _):
        pltpu.sync_copy(W.at[pl.ds(0, 25)], acc.at[pl.ds(nbase + k * 25, 25)])
        return 0
    lax.fori_loop(0, NSLICE // 25, _zcp, 0)

    dbase = sid * DSLICE

    def _zcpd(k, _):
        pltpu.sync_copy(exb, dacc.at[pl.ds(dbase + k * CHUNK, CHUNK)])
        return 0
    lax.fori_loop(0, DSLICE // CHUNK, _zcpd, 0)

    pltpu.sync_copy(vecs_hbm, vecs)
    we8 = [vecs[0, pl.ds(k * 16, 16)] for k in range(8)]
    at8 = [vecs[1, pl.ds(k * 16, 16)] for k in range(8)]
    plsc.subcore_barrier()

    def _issue(jb, bufA, bufB, semA, semB):
        a = pltpu.async_copy(xl_hbm.at[srci.at[jb]], bufA, semA)
        b = pltpu.async_copy(xr_hbm.at[dsti.at[jb]], bufB, semB)
        return a, b

    def _compute(i, jb, bufA, bufB, semA, semB):
        pltpu.make_async_copy(xl_hbm.at[srci.at[jb]], bufA, semA).wait()
        pltpu.make_async_copy(xr_hbm.at[dsti.at[jb]], bufB, semB).wait()

        jbs = jnp.full((16,), jb, jnp.int32)

        gsems = (ssem0, ssem1, ssem2, ssem3)
        for g in range(NGRP):
            @pl.when(jb != 0)
            def _():
                pltpu.make_async_copy(W.at[pl.ds(g * 16, 16)],
                                      acc.at[dsti4.at[0]], gsems[g]).wait()

            if g == 0:
                @pl.when(jb != 0)
                def _():
                    pltpu.make_async_copy(exb, dacc.at[dsti.at[jb]],
                                          dsem).wait()
            av = plsc.load_gather(attrv, [jbs, rows16 + g * 16])

            @plsc.parallel_loop(0, 16, unroll=2)
            def _edge(e):
                ge = e + g * 16
                ab = av.at[jnp.full((16,), e, jnp.int32)].get(
                    mode="promise_in_bounds")
                arow = [bufA[ge, pl.ds(k * 16, 16)] for k in range(8)]
                accs = []
                for k in range(8):
                    m = arow[k] + bufB[ge, pl.ds(k * 16, 16)] + ab * we8[k]
                    m = jnp.where(m > 0, m, 0.2 * m)
                    accs.append(m * at8[k])
                vacc = (((accs[0] + accs[1]) + (accs[2] + accs[3]))
                        + ((accs[4] + accs[5]) + (accs[6] + accs[7])))
                exv = jnp.exp(jnp.full((16,), jnp.sum(vacc)))
                for k in range(8):
                    W[ge, pl.ds(k * 16, 16)] = arow[k] * exv
                plsc.store_scatter(exb, [jnp.full((16,), ge, jnp.int32)],
                                   exv, mask=rows16 == e)

            pltpu.async_copy(W.at[pl.ds(g * 16, 16)],
                             acc.at[dsti4.at[jb * 4 + g]], gsems[g],
                             add=True)
        pltpu.async_copy(exb, dacc.at[dsti.at[jb]], dsem, add=True)

    def _chunk(i, _):
        r = start + i
        jb = i % 16

        @pl.when(jb == 0)
        def _():
            @pl.when(i > 0)
            def _():
                for g, sg in enumerate((ssem0, ssem1, ssem2, ssem3)):
                    pltpu.make_async_copy(
                        W.at[pl.ds(g * 16, 16)],
                        acc.at[dsti4.at[0]], sg).wait()
                pltpu.make_async_copy(exb, dacc.at[dsti.at[0]], dsem).wait()

            pltpu.sync_copy(src_hbm.at[pl.ds(r, 16)], srci)
            pltpu.sync_copy(dst_hbm.at[pl.ds(r, 16)], dsti)
            pltpu.sync_copy(dst4_hbm.at[pl.ds(r * 4, 64)], dsti4)
            pltpu.sync_copy(attr_hbm.at[pl.ds(r, 16)], attrv)

            @pl.when(i % 2 == 0)
            def _():
                _issue(jb, A0, B0, ga0, gb0)

            @pl.when(i % 2 == 1)
            def _():
                _issue(jb, A1, B1, ga1, gb1)

        @pl.when((i + 1 < nch) & (jb != 15))
        def _():
            @pl.when(i % 2 == 0)
            def _():
                _issue(jb + 1, A1, B1, ga1, gb1)

            @pl.when(i % 2 == 1)
            def _():
                _issue(jb + 1, A0, B0, ga0, gb0)

        @pl.when(i % 2 == 0)
        def _():
            _compute(i, jb, A0, B0, ga0, gb0)

        @pl.when(i % 2 == 1)
        def _():
            _compute(i, jb, A1, B1, ga1, gb1)
        return 0

    lax.fori_loop(0, nch, _chunk, 0)

    for g, sg in enumerate((ssem0, ssem1, ssem2, ssem3)):
        pltpu.make_async_copy(W.at[pl.ds(g * 16, 16)],
                              acc.at[dsti4.at[0]], sg).wait()
    pltpu.make_async_copy(exb, dacc.at[dsti.at[0]], dsem).wait()

    plsc.subcore_barrier()

    def _ocp(k, _):
        pltpu.sync_copy(acc.at[pl.ds(nbase + k * 125, 125)],
                        num_hbm.at[cid, pl.ds(nbase + k * 125, 125)])
        return 0
    lax.fori_loop(0, NSLICE // 125, _ocp, 0)
    pltpu.sync_copy(dacc.at[pl.ds(dbase, DSLICE)],
                    den_hbm.at[cid, pl.ds(dbase, DSLICE)])


def _edge_phase(xl, xr, src2d, dst2d, dst4, attr2d, vecs):
    mesh = plsc.VectorSubcoreMesh(core_axis_name="c", subcore_axis_name="s")
    f = pl.kernel(
        _edge_body,
        out_type=[jax.ShapeDtypeStruct((NC, NPAD, D), jnp.float32),
                  jax.ShapeDtypeStruct((NC, DPAD), jnp.float32)],
        mesh=mesh,
        compiler_params=pltpu.CompilerParams(
            needs_layout_passes=False, use_tc_tiling_on_sc=False),
        scratch_types=[
            pltpu.VMEM_SHARED((NPAD, D), jnp.float32),
            pltpu.VMEM_SHARED((DPAD,), jnp.float32),
            pltpu.VMEM((16, CHUNK), jnp.int32),
            pltpu.VMEM((16, CHUNK), jnp.int32),
            pltpu.VMEM((64, 16), jnp.int32),
            pltpu.VMEM((16, CHUNK), jnp.float32),
            pltpu.VMEM((CHUNK, D), jnp.float32),
            pltpu.VMEM((CHUNK, D), jnp.float32),
            pltpu.VMEM((CHUNK, D), jnp.float32),
            pltpu.VMEM((CHUNK, D), jnp.float32),
            pltpu.VMEM((CHUNK, D), jnp.float32),
            pltpu.VMEM((CHUNK,), jnp.float32),
            pltpu.VMEM((2, D), jnp.float32),
            pltpu.SemaphoreType.DMA,
            pltpu.SemaphoreType.DMA,
            pltpu.SemaphoreType.DMA,
            pltpu.SemaphoreType.DMA,
            pltpu.SemaphoreType.DMA,
            pltpu.SemaphoreType.DMA,
            pltpu.SemaphoreType.DMA,
            pltpu.SemaphoreType.DMA,
            pltpu.SemaphoreType.DMA,
        ],
    )
    return f(xl, xr, src2d, dst2d, dst4, attr2d, vecs)


def _dec_body(u_ref, dn_ref, bias_ref, Wd0_ref, bd0_ref, Wd1_ref, bd1_ref,
              out_ref):
    num = u_ref[0] + u_ref[1]
    den = dn_ref[0, :, :] + dn_ref[1, :, :]
    agg = num / (den + 1e-16) + bias_ref[...]
    z = jnp.maximum(agg, 0.0)
    d = jnp.maximum(z @ Wd0_ref[...] + bd0_ref[...], 0.0)
    logits = d @ Wd1_ref[...] + bd1_ref[...]
    out_ref[...] = jax.nn.softmax(logits, axis=-1)


def _decoder(u, dens, bias_g, Wd0, bd0, Wd1, bd1):
    n = N_NODES
    grid = (n // N_ROW_BLK,)
    rep = lambda i: (0, 0)
    return pl.pallas_call(
        _dec_body,
        grid=grid,
        in_specs=[
            pl.BlockSpec((NC, N_ROW_BLK, D), lambda i: (0, i, 0)),
            pl.BlockSpec((NC, N_ROW_BLK, 1), lambda i: (0, i, 0)),
            pl.BlockSpec((128,), lambda i: (0,)),
            pl.BlockSpec((128, 64), rep),
            pl.BlockSpec((64,), lambda i: (0,)),
            pl.BlockSpec((64, 30), rep),
            pl.BlockSpec((30,), lambda i: (0,)),
        ],
        out_specs=pl.BlockSpec((N_ROW_BLK, 30), lambda i: (i, 0)),
        out_shape=jax.ShapeDtypeStruct((n, 30), jnp.float32),
    )(u, dens[..., None], bias_g, Wd0, bd0, Wd1, bd1)


def kernel(x, edge_index, edge_attr, pos, W0, b0, W1, b1, W2, b2, Wl, bl,
           Wr, br, We, att, bias_g, Wd0, bd0, Wd1, bd1):
    xl, xr = _encoder(x, pos, W0, b0, W1, b1, W2, b2, Wl, bl, Wr, br)
    pad = ((0, IDX_PAD), (0, 0))
    src2d = jnp.pad(edge_index[0].reshape(IDX_ROWS, CHUNK), pad)
    dst2d = jnp.pad(edge_index[1].reshape(IDX_ROWS, CHUNK), pad)
    attr2d = jnp.pad(edge_attr.reshape(IDX_ROWS, CHUNK), pad)
    vecs = jnp.stack([We[0], att])
    dst4 = jnp.pad(edge_index[1].reshape(IDX_ROWS * 4, 16),
                   ((0, IDX_PAD * 4), (0, 0)))
    u, dens = _edge_phase(xl, xr, src2d, dst2d, dst4, attr2d, vecs)
    return _decoder(u, dens, bias_g, Wd0, bd0, Wd1, bd1)
